# baseline (speedup 1.0000x reference)
import functools

import jax
import jax.numpy as jnp
import numpy as np
from jax import lax
from jax.experimental import pallas as pl
from jax.experimental.pallas import tpu as pltpu
from jax.experimental.pallas import tpu_sc as plsc

NC = 2
NS = 16
L = 16
NW = NC * NS

DESC = 512
WORDS = DESC // 4
PIX = 1024
BATCH = 32


def _brief_body(n_per_w, n_batches, patches_hbm, idx1_hbm, idx2_hbm, out_hbm,
                idx1_v, idx2_v, buf_v, out_v):
    wid = lax.axis_index("s") * NC + lax.axis_index("c")
    base = wid * n_per_w

    pltpu.sync_copy(idx1_hbm, idx1_v)
    pltpu.sync_copy(idx2_hbm, idx2_v)

    def batch_body(b, _):
        p0 = base + b * BATCH
        pltpu.sync_copy(patches_hbm.at[pl.ds(p0 * PIX, BATCH * PIX)], buf_v)

        def group_body(g, _):
            g0 = g * 64
            i1 = [idx1_v[pl.ds(g0 + c * L, L)] for c in range(4)]
            i2 = [idx2_v[pl.ds(g0 + c * L, L)] for c in range(4)]

            def patch_body(p, _):
                off = p * PIX
                w = jnp.zeros((L,), jnp.int32)
                for c in range(4):
                    v1 = plsc.load_gather(buf_v, [i1[c] + off])
                    v2 = plsc.load_gather(buf_v, [i2[c] + off])
                    w = w | jnp.where(v1 < v2, jnp.int32(1 << (8 * c)),
                                      jnp.int32(0))
                out_v[pl.ds(p * WORDS + g * L, L)] = w
                return 0

            lax.fori_loop(0, BATCH, patch_body, 0)
            return 0

        lax.fori_loop(0, 8, group_body, 0)
        pltpu.sync_copy(out_v, out_hbm.at[pl.ds(p0 * WORDS, BATCH * WORDS)])
        return 0

    lax.fori_loop(0, n_batches, batch_body, 0)


def kernel(patches, pos1, pos2):
    n = patches.shape[0]
    assert n % (NW * BATCH) == 0
    n_per_w = n // NW
    n_batches = n_per_w // BATCH

    idx1 = (pos1[:, 0] * 32 + pos1[:, 1]).astype(jnp.int32)
    idx2 = (pos2[:, 0] * 32 + pos2[:, 1]).astype(jnp.int32)
    perm = np.arange(DESC).reshape(8, 16, 4).transpose(0, 2, 1).reshape(-1)
    idx1 = idx1[perm]
    idx2 = idx2[perm]

    pflat = patches.reshape(n * PIX)

    mesh = plsc.VectorSubcoreMesh(core_axis_name="c", subcore_axis_name="s")
    body = functools.partial(_brief_body, n_per_w, n_batches)
    out_words = pl.kernel(
        body,
        out_type=jax.ShapeDtypeStruct((n * WORDS,), jnp.int32),
        mesh=mesh,
        scratch_types=[
            pltpu.VMEM((DESC,), jnp.int32),
            pltpu.VMEM((DESC,), jnp.int32),
            pltpu.VMEM((BATCH * PIX,), jnp.float32),
            pltpu.VMEM((BATCH * WORDS,), jnp.int32),
        ],
        compiler_params=pltpu.CompilerParams(needs_layout_passes=False),
    )(pflat, idx1, idx2)

    out_bytes = lax.bitcast_convert_type(out_words.reshape(n, WORDS),
                                         jnp.uint8)
    return out_bytes.reshape(n, DESC).astype(jnp.bool_)
